# Initial kernel scaffold; baseline (speedup 1.0000x reference)
import jax
import jax.numpy as jnp
from jax.experimental import pallas as pl


def kernel(x, edge_index, W1, b1, W2, b2):
    raise NotImplementedError("write your pallas kernel here")


# speedup vs baseline: 5.8988x; 5.8988x over previous
import functools

import jax
import jax.numpy as jnp
from jax import lax
from jax.experimental import pallas as pl
from jax.experimental.pallas import tpu as pltpu
from jax.experimental.pallas import tpu_sc as plsc

N = 10000
E = 320000
DF = 128
DH = 128
NCLS = 64

SC_CORES = 2
SC_TILES = 16
BATCH = 100
EDGES_PER_TILE = E // SC_TILES
BATCHES_PER_TILE = EDGES_PER_TILE // BATCH
N_PAD = 10240
ROWS_PER_TILE = N_PAD // SC_TILES
DEG_W = 16

_MESH = dict(core_axis_name="c", subcore_axis_name="s",
             num_cores=SC_CORES, num_subcores=SC_TILES)


def _sc_agg_body(with_deg, width, *refs):
    if with_deg:
        (h0, h1, src2, dst2, z_w, z_deg, ones_h,
         agg0, agg1, deg0, deg1,
         src_v, dst_v, rows_v, accum, sem, ones_v, dega) = refs
    else:
        (h0, h1, src2, dst2, z_w,
         agg0, agg1,
         src_v, dst_v, rows_v, accum, sem) = refs

    c = lax.axis_index("c")
    t = lax.axis_index("s")
    rsl = pl.ds(t * ROWS_PER_TILE, ROWS_PER_TILE)

    pltpu.sync_copy(src2.at[pl.ds(t * BATCHES_PER_TILE, BATCHES_PER_TILE)], src_v)
    pltpu.sync_copy(dst2.at[pl.ds(t * BATCHES_PER_TILE, BATCHES_PER_TILE)], dst_v)
    pltpu.sync_copy(z_w.at[rsl], accum.at[rsl])
    if with_deg:
        pltpu.sync_copy(z_deg.at[rsl], dega.at[rsl])
        pltpu.sync_copy(ones_h, ones_v)
    plsc.subcore_barrier()

    half = BATCHES_PER_TILE // 2

    def step(j, carry):
        sidx = src_v.at[j]
        didx = dst_v.at[j]

        @pl.when(c == 0)
        def _():
            pltpu.async_copy(h0.at[sidx], rows_v, sem).wait()

        @pl.when(c == 1)
        def _():
            pltpu.async_copy(h1.at[sidx], rows_v, sem).wait()

        pltpu.sync_copy(rows_v, accum.at[didx], add=True)
        if with_deg:
            @pl.when((j < half) == (c == 0))
            def _():
                pltpu.sync_copy(ones_v, dega.at[didx], add=True)
        return carry

    lax.fori_loop(0, BATCHES_PER_TILE, step, 0)
    plsc.subcore_barrier()

    @pl.when(c == 0)
    def _():
        pltpu.sync_copy(accum.at[rsl], agg0.at[rsl])
        if with_deg:
            pltpu.sync_copy(dega.at[rsl], deg0.at[rsl])

    @pl.when(c == 1)
    def _():
        pltpu.sync_copy(accum.at[rsl], agg1.at[rsl])
        if with_deg:
            pltpu.sync_copy(dega.at[rsl], deg1.at[rsl])


def _make_sc_agg(width, with_deg):
    out_type = [jax.ShapeDtypeStruct((N_PAD, width), jnp.float32),
                jax.ShapeDtypeStruct((N_PAD, width), jnp.float32)]
    scratch = [
        pltpu.VMEM((BATCHES_PER_TILE, BATCH), jnp.int32),
        pltpu.VMEM((BATCHES_PER_TILE, BATCH), jnp.int32),
        pltpu.VMEM((BATCH, width), jnp.float32),
        pltpu.VMEM_SHARED((N_PAD, width), jnp.float32),
        pltpu.SemaphoreType.DMA,
    ]
    if with_deg:
        out_type += [jax.ShapeDtypeStruct((N_PAD, DEG_W), jnp.float32),
                     jax.ShapeDtypeStruct((N_PAD, DEG_W), jnp.float32)]
        scratch += [pltpu.VMEM((BATCH, DEG_W), jnp.float32),
                    pltpu.VMEM_SHARED((N_PAD, DEG_W), jnp.float32)]
    return pl.kernel(
        functools.partial(_sc_agg_body, with_deg, width),
        out_type=out_type,
        mesh=plsc.VectorSubcoreMesh(**_MESH),
        scratch_types=scratch,
        compiler_params=pltpu.CompilerParams(use_tc_tiling_on_sc=False),
    )


def _mm1_body(x_ref, w_ref, o0, o1):
    h = jnp.dot(x_ref[...], w_ref[...], preferred_element_type=jnp.float32)
    o0[...] = h[:, : DH // 2]
    o1[...] = h[:, DH // 2:]


def _mid_body(a0, a1, d0, d1, b1r, w2r, o0, o1):
    deg = jnp.maximum(d0[:, 0:1] + d1[:, 0:1], 1.0)
    a = jnp.concatenate([a0[...], a1[...]], axis=1)
    h1 = jnp.maximum(a / deg + b1r[...], 0.0)
    h2 = jnp.dot(h1, w2r[...], preferred_element_type=jnp.float32)
    o0[...] = h2[:, : NCLS // 2]
    o1[...] = h2[:, NCLS // 2:]


def _out_body(a0, a1, d0, d1, b2r, o):
    deg = jnp.maximum(d0[:, 0:1] + d1[:, 0:1], 1.0)
    z = jnp.concatenate([a0[...], a1[...]], axis=1) / deg + b2r[...]
    m = jnp.max(z, axis=1, keepdims=True)
    lse = jnp.log(jnp.sum(jnp.exp(z - m), axis=1, keepdims=True))
    o[...] = z - m - lse


_R = 1000


def _mm1(x, W1):
    return pl.pallas_call(
        _mm1_body,
        grid=(N // _R,),
        in_specs=[pl.BlockSpec((_R, DF), lambda i: (i, 0)),
                  pl.BlockSpec((DF, DH), lambda i: (0, 0))],
        out_specs=[pl.BlockSpec((_R, DH // 2), lambda i: (i, 0)),
                   pl.BlockSpec((_R, DH // 2), lambda i: (i, 0))],
        out_shape=[jax.ShapeDtypeStruct((N, DH // 2), jnp.float32)] * 2,
    )(x, W1)


def _mid(a0, a1, d0, d1, b1r, W2):
    return pl.pallas_call(
        _mid_body,
        grid=(N // _R,),
        in_specs=[pl.BlockSpec((_R, DH // 2), lambda i: (i, 0)),
                  pl.BlockSpec((_R, DH // 2), lambda i: (i, 0)),
                  pl.BlockSpec((_R, DEG_W), lambda i: (i, 0)),
                  pl.BlockSpec((_R, DEG_W), lambda i: (i, 0)),
                  pl.BlockSpec((1, DH), lambda i: (0, 0)),
                  pl.BlockSpec((DH, NCLS), lambda i: (0, 0))],
        out_specs=[pl.BlockSpec((_R, NCLS // 2), lambda i: (i, 0)),
                   pl.BlockSpec((_R, NCLS // 2), lambda i: (i, 0))],
        out_shape=[jax.ShapeDtypeStruct((N, NCLS // 2), jnp.float32)] * 2,
    )(a0, a1, d0, d1, b1r, W2)


def _out(a0, a1, d0, d1, b2r):
    return pl.pallas_call(
        _out_body,
        grid=(N // _R,),
        in_specs=[pl.BlockSpec((_R, NCLS // 2), lambda i: (i, 0)),
                  pl.BlockSpec((_R, NCLS // 2), lambda i: (i, 0)),
                  pl.BlockSpec((_R, DEG_W), lambda i: (i, 0)),
                  pl.BlockSpec((_R, DEG_W), lambda i: (i, 0)),
                  pl.BlockSpec((1, NCLS), lambda i: (0, 0))],
        out_specs=pl.BlockSpec((_R, NCLS), lambda i: (i, 0)),
        out_shape=jax.ShapeDtypeStruct((N, NCLS), jnp.float32),
    )(a0, a1, d0, d1, b2r)


_SC_KERNEL_CACHE = {}


def _sc_agg(width, with_deg):
    key = (width, with_deg)
    if key not in _SC_KERNEL_CACHE:
        _SC_KERNEL_CACHE[key] = _make_sc_agg(width, with_deg)
    return _SC_KERNEL_CACHE[key]


def kernel(x, edge_index, W1, b1, W2, b2):
    src2 = edge_index[0].reshape(E // BATCH, BATCH)
    dst2 = edge_index[1].reshape(E // BATCH, BATCH)
    z64 = jnp.zeros((N_PAD, DH // 2), jnp.float32)
    z32 = jnp.zeros((N_PAD, NCLS // 2), jnp.float32)
    zdeg = jnp.zeros((N_PAD, DEG_W), jnp.float32)
    ones = jnp.ones((BATCH, DEG_W), jnp.float32)

    h0, h1 = _mm1(x, W1)
    agg0, agg1, deg0, deg1 = _sc_agg(DH // 2, True)(
        h0, h1, src2, dst2, z64, zdeg, ones)
    g0, g1 = _mid(agg0, agg1, deg0, deg1, b1.reshape(1, DH), W2)
    s0, s1 = _sc_agg(NCLS // 2, False)(g0, g1, src2, dst2, z32)
    return _out(s0, s1, deg0, deg1, b2.reshape(1, NCLS))


# speedup vs baseline: 9.2492x; 1.5680x over previous
import functools

import jax
import jax.numpy as jnp
from jax import lax
from jax.experimental import pallas as pl
from jax.experimental.pallas import tpu as pltpu
from jax.experimental.pallas import tpu_sc as plsc

N = 10000
E = 320000
DF = 128
DH = 128
NCLS = 64

SC_CORES = 2
SC_TILES = 16
BATCH = 400
EDGES_PER_TILE = E // SC_TILES
BATCHES_PER_TILE = EDGES_PER_TILE // BATCH
N_PAD = 10240
ROWS_PER_TILE = N_PAD // SC_TILES
DEG_W = 16

_MESH = dict(core_axis_name="c", subcore_axis_name="s",
             num_cores=SC_CORES, num_subcores=SC_TILES)


def _sc_agg_body(with_deg, width, *refs):
    if with_deg:
        (h0, h1, src2, dst2, z_w, z_deg, ones_h,
         agg0, agg1, deg0, deg1,
         src_v, dst_v, rows_v, accum, sem, ones_v, dega) = refs
    else:
        (h0, h1, src2, dst2, z_w,
         agg0, agg1,
         src_v, dst_v, rows_v, accum, sem) = refs

    c = lax.axis_index("c")
    t = lax.axis_index("s")
    rsl = pl.ds(t * ROWS_PER_TILE, ROWS_PER_TILE)

    pltpu.sync_copy(src2.at[pl.ds(t * BATCHES_PER_TILE, BATCHES_PER_TILE)], src_v)
    pltpu.sync_copy(dst2.at[pl.ds(t * BATCHES_PER_TILE, BATCHES_PER_TILE)], dst_v)
    pltpu.sync_copy(z_w.at[rsl], accum.at[rsl])
    if with_deg:
        pltpu.sync_copy(z_deg.at[rsl], dega.at[rsl])
        pltpu.sync_copy(ones_h, ones_v)
    plsc.subcore_barrier()

    half = BATCHES_PER_TILE // 2

    def step(j, carry):
        sidx = src_v.at[j]
        didx = dst_v.at[j]

        @pl.when(c == 0)
        def _():
            pltpu.async_copy(h0.at[sidx], rows_v, sem).wait()

        @pl.when(c == 1)
        def _():
            pltpu.async_copy(h1.at[sidx], rows_v, sem).wait()

        pltpu.sync_copy(rows_v, accum.at[didx], add=True)
        if with_deg:
            @pl.when((j < half) == (c == 0))
            def _():
                pltpu.sync_copy(ones_v, dega.at[didx], add=True)
        return carry

    lax.fori_loop(0, BATCHES_PER_TILE, step, 0)
    plsc.subcore_barrier()

    @pl.when(c == 0)
    def _():
        pltpu.sync_copy(accum.at[rsl], agg0.at[rsl])
        if with_deg:
            pltpu.sync_copy(dega.at[rsl], deg0.at[rsl])

    @pl.when(c == 1)
    def _():
        pltpu.sync_copy(accum.at[rsl], agg1.at[rsl])
        if with_deg:
            pltpu.sync_copy(dega.at[rsl], deg1.at[rsl])


def _make_sc_agg(width, with_deg):
    out_type = [jax.ShapeDtypeStruct((N_PAD, width), jnp.float32),
                jax.ShapeDtypeStruct((N_PAD, width), jnp.float32)]
    scratch = [
        pltpu.VMEM((BATCHES_PER_TILE, BATCH), jnp.int32),
        pltpu.VMEM((BATCHES_PER_TILE, BATCH), jnp.int32),
        pltpu.VMEM((BATCH, width), jnp.float32),
        pltpu.VMEM_SHARED((N_PAD, width), jnp.float32),
        pltpu.SemaphoreType.DMA,
    ]
    if with_deg:
        out_type += [jax.ShapeDtypeStruct((N_PAD, DEG_W), jnp.float32),
                     jax.ShapeDtypeStruct((N_PAD, DEG_W), jnp.float32)]
        scratch += [pltpu.VMEM((BATCH, DEG_W), jnp.float32),
                    pltpu.VMEM_SHARED((N_PAD, DEG_W), jnp.float32)]
    return pl.kernel(
        functools.partial(_sc_agg_body, with_deg, width),
        out_type=out_type,
        mesh=plsc.VectorSubcoreMesh(**_MESH),
        scratch_types=scratch,
        compiler_params=pltpu.CompilerParams(use_tc_tiling_on_sc=False),
    )


def _mm1_body(x_ref, w_ref, o0, o1):
    h = jnp.dot(x_ref[...], w_ref[...], preferred_element_type=jnp.float32)
    o0[...] = h[:, : DH // 2]
    o1[...] = h[:, DH // 2:]


def _mid_body(a0, a1, d0, d1, b1r, w2r, o0, o1):
    deg = jnp.maximum(d0[:, 0:1] + d1[:, 0:1], 1.0)
    a = jnp.concatenate([a0[...], a1[...]], axis=1)
    h1 = jnp.maximum(a / deg + b1r[...], 0.0)
    h2 = jnp.dot(h1, w2r[...], preferred_element_type=jnp.float32)
    o0[...] = h2[:, : NCLS // 2]
    o1[...] = h2[:, NCLS // 2:]


def _out_body(a0, a1, d0, d1, b2r, o):
    deg = jnp.maximum(d0[:, 0:1] + d1[:, 0:1], 1.0)
    z = jnp.concatenate([a0[...], a1[...]], axis=1) / deg + b2r[...]
    m = jnp.max(z, axis=1, keepdims=True)
    lse = jnp.log(jnp.sum(jnp.exp(z - m), axis=1, keepdims=True))
    o[...] = z - m - lse


_R = 1000


def _mm1(x, W1):
    return pl.pallas_call(
        _mm1_body,
        grid=(N // _R,),
        in_specs=[pl.BlockSpec((_R, DF), lambda i: (i, 0)),
                  pl.BlockSpec((DF, DH), lambda i: (0, 0))],
        out_specs=[pl.BlockSpec((_R, DH // 2), lambda i: (i, 0)),
                   pl.BlockSpec((_R, DH // 2), lambda i: (i, 0))],
        out_shape=[jax.ShapeDtypeStruct((N, DH // 2), jnp.float32)] * 2,
    )(x, W1)


def _mid(a0, a1, d0, d1, b1r, W2):
    return pl.pallas_call(
        _mid_body,
        grid=(N // _R,),
        in_specs=[pl.BlockSpec((_R, DH // 2), lambda i: (i, 0)),
                  pl.BlockSpec((_R, DH // 2), lambda i: (i, 0)),
                  pl.BlockSpec((_R, DEG_W), lambda i: (i, 0)),
                  pl.BlockSpec((_R, DEG_W), lambda i: (i, 0)),
                  pl.BlockSpec((1, DH), lambda i: (0, 0)),
                  pl.BlockSpec((DH, NCLS), lambda i: (0, 0))],
        out_specs=[pl.BlockSpec((_R, NCLS // 2), lambda i: (i, 0)),
                   pl.BlockSpec((_R, NCLS // 2), lambda i: (i, 0))],
        out_shape=[jax.ShapeDtypeStruct((N, NCLS // 2), jnp.float32)] * 2,
    )(a0, a1, d0, d1, b1r, W2)


def _out(a0, a1, d0, d1, b2r):
    return pl.pallas_call(
        _out_body,
        grid=(N // _R,),
        in_specs=[pl.BlockSpec((_R, NCLS // 2), lambda i: (i, 0)),
                  pl.BlockSpec((_R, NCLS // 2), lambda i: (i, 0)),
                  pl.BlockSpec((_R, DEG_W), lambda i: (i, 0)),
                  pl.BlockSpec((_R, DEG_W), lambda i: (i, 0)),
                  pl.BlockSpec((1, NCLS), lambda i: (0, 0))],
        out_specs=pl.BlockSpec((_R, NCLS), lambda i: (i, 0)),
        out_shape=jax.ShapeDtypeStruct((N, NCLS), jnp.float32),
    )(a0, a1, d0, d1, b2r)


_SC_KERNEL_CACHE = {}


def _sc_agg(width, with_deg):
    key = (width, with_deg)
    if key not in _SC_KERNEL_CACHE:
        _SC_KERNEL_CACHE[key] = _make_sc_agg(width, with_deg)
    return _SC_KERNEL_CACHE[key]


def kernel(x, edge_index, W1, b1, W2, b2):
    src2 = edge_index[0].reshape(E // BATCH, BATCH)
    dst2 = edge_index[1].reshape(E // BATCH, BATCH)
    z64 = jnp.zeros((N_PAD, DH // 2), jnp.float32)
    z32 = jnp.zeros((N_PAD, NCLS // 2), jnp.float32)
    zdeg = jnp.zeros((N_PAD, DEG_W), jnp.float32)
    ones = jnp.ones((BATCH, DEG_W), jnp.float32)

    h0, h1 = _mm1(x, W1)
    agg0, agg1, deg0, deg1 = _sc_agg(DH // 2, True)(
        h0, h1, src2, dst2, z64, zdeg, ones)
    g0, g1 = _mid(agg0, agg1, deg0, deg1, b1.reshape(1, DH), W2)
    s0, s1 = _sc_agg(NCLS // 2, False)(g0, g1, src2, dst2, z32)
    return _out(s0, s1, deg0, deg1, b2.reshape(1, NCLS))


# speedup vs baseline: 10.4315x; 1.1278x over previous
import functools

import jax
import jax.numpy as jnp
from jax import lax
from jax.experimental import pallas as pl
from jax.experimental.pallas import tpu as pltpu
from jax.experimental.pallas import tpu_sc as plsc

N = 10000
E = 320000
DF = 128
DH = 128
NCLS = 64

SC_CORES = 2
SC_TILES = 16
BATCH = 500
EDGES_PER_TILE = E // SC_TILES
BATCHES_PER_TILE = EDGES_PER_TILE // BATCH
N_PAD = 10240
ROWS_PER_TILE = N_PAD // SC_TILES
DEG_W = 16

_MESH = dict(core_axis_name="c", subcore_axis_name="s",
             num_cores=SC_CORES, num_subcores=SC_TILES)


def _sc_agg_body(with_deg, width, *refs):
    if with_deg:
        (h0, h1, src2, dst2, z_w, z_deg, ones_h,
         agg0, agg1, deg0, deg1,
         src_a, src_b, dst_a, dst_b, rows_a, rows_b, accum, sem_a, sem_b,
         ones_v, dega) = refs
    else:
        (h0, h1, src2, dst2, z_w,
         agg0, agg1,
         src_a, src_b, dst_a, dst_b, rows_a, rows_b, accum, sem_a, sem_b) = refs

    c = lax.axis_index("c")
    t = lax.axis_index("s")
    rsl = pl.ds(t * ROWS_PER_TILE, ROWS_PER_TILE)
    nb = BATCHES_PER_TILE
    half = nb // 2

    def load_idx(j, src_v, dst_v):
        row = t * nb + j
        pltpu.sync_copy(src2.at[row], src_v)
        pltpu.sync_copy(dst2.at[row], dst_v)

    def start_gather(src_v, rows_v, sem):
        @pl.when(c == 0)
        def _():
            pltpu.async_copy(h0.at[src_v], rows_v, sem)

        @pl.when(c == 1)
        def _():
            pltpu.async_copy(h1.at[src_v], rows_v, sem)

    def wait_gather(rows_v, sem):
        pltpu.make_async_copy(h0.at[pl.ds(0, BATCH)], rows_v, sem).wait()

    def commit(j, dst_v, rows_v):
        pltpu.sync_copy(rows_v, accum.at[dst_v], add=True)
        if with_deg:
            @pl.when((j < half) == (c == 0))
            def _():
                pltpu.sync_copy(ones_v, dega.at[dst_v], add=True)

    pltpu.sync_copy(z_w.at[rsl], accum.at[rsl])
    if with_deg:
        pltpu.sync_copy(z_deg.at[rsl], dega.at[rsl])
        pltpu.sync_copy(ones_h, ones_v)
    plsc.subcore_barrier()

    load_idx(0, src_a, dst_a)
    start_gather(src_a, rows_a, sem_a)

    def step(i, carry):
        j0 = 2 * i
        load_idx(j0 + 1, src_b, dst_b)
        start_gather(src_b, rows_b, sem_b)
        wait_gather(rows_a, sem_a)
        commit(j0, dst_a, rows_a)

        @pl.when(i + 1 < nb // 2)
        def _():
            load_idx(j0 + 2, src_a, dst_a)
            start_gather(src_a, rows_a, sem_a)

        wait_gather(rows_b, sem_b)
        commit(j0 + 1, dst_b, rows_b)
        return carry

    lax.fori_loop(0, nb // 2, step, 0)
    plsc.subcore_barrier()

    @pl.when(c == 0)
    def _():
        pltpu.sync_copy(accum.at[rsl], agg0.at[rsl])
        if with_deg:
            pltpu.sync_copy(dega.at[rsl], deg0.at[rsl])

    @pl.when(c == 1)
    def _():
        pltpu.sync_copy(accum.at[rsl], agg1.at[rsl])
        if with_deg:
            pltpu.sync_copy(dega.at[rsl], deg1.at[rsl])


def _make_sc_agg(width, with_deg):
    out_type = [jax.ShapeDtypeStruct((N_PAD, width), jnp.float32),
                jax.ShapeDtypeStruct((N_PAD, width), jnp.float32)]
    scratch = [
        pltpu.VMEM((BATCH,), jnp.int32),
        pltpu.VMEM((BATCH,), jnp.int32),
        pltpu.VMEM((BATCH,), jnp.int32),
        pltpu.VMEM((BATCH,), jnp.int32),
        pltpu.VMEM((BATCH, width), jnp.float32),
        pltpu.VMEM((BATCH, width), jnp.float32),
        pltpu.VMEM_SHARED((N_PAD, width), jnp.float32),
        pltpu.SemaphoreType.DMA,
        pltpu.SemaphoreType.DMA,
    ]
    if with_deg:
        out_type += [jax.ShapeDtypeStruct((N_PAD, DEG_W), jnp.float32),
                     jax.ShapeDtypeStruct((N_PAD, DEG_W), jnp.float32)]
        scratch += [pltpu.VMEM((BATCH, DEG_W), jnp.float32),
                    pltpu.VMEM_SHARED((N_PAD, DEG_W), jnp.float32)]
    return pl.kernel(
        functools.partial(_sc_agg_body, with_deg, width),
        out_type=out_type,
        mesh=plsc.VectorSubcoreMesh(**_MESH),
        scratch_types=scratch,
        compiler_params=pltpu.CompilerParams(use_tc_tiling_on_sc=False),
    )


def _mm1_body(x_ref, w_ref, o0, o1):
    h = jnp.dot(x_ref[...], w_ref[...], preferred_element_type=jnp.float32)
    o0[...] = h[:, : DH // 2]
    o1[...] = h[:, DH // 2:]


def _mid_body(a0, a1, d0, d1, b1r, w2r, o0, o1):
    deg = jnp.maximum(d0[:, 0:1] + d1[:, 0:1], 1.0)
    a = jnp.concatenate([a0[...], a1[...]], axis=1)
    h1 = jnp.maximum(a / deg + b1r[...], 0.0)
    h2 = jnp.dot(h1, w2r[...], preferred_element_type=jnp.float32)
    o0[...] = h2[:, : NCLS // 2]
    o1[...] = h2[:, NCLS // 2:]


def _out_body(a0, a1, d0, d1, b2r, o):
    deg = jnp.maximum(d0[:, 0:1] + d1[:, 0:1], 1.0)
    z = jnp.concatenate([a0[...], a1[...]], axis=1) / deg + b2r[...]
    m = jnp.max(z, axis=1, keepdims=True)
    lse = jnp.log(jnp.sum(jnp.exp(z - m), axis=1, keepdims=True))
    o[...] = z - m - lse


_R = 1000


def _mm1(x, W1):
    return pl.pallas_call(
        _mm1_body,
        grid=(N // _R,),
        in_specs=[pl.BlockSpec((_R, DF), lambda i: (i, 0)),
                  pl.BlockSpec((DF, DH), lambda i: (0, 0))],
        out_specs=[pl.BlockSpec((_R, DH // 2), lambda i: (i, 0)),
                   pl.BlockSpec((_R, DH // 2), lambda i: (i, 0))],
        out_shape=[jax.ShapeDtypeStruct((N, DH // 2), jnp.float32)] * 2,
    )(x, W1)


def _mid(a0, a1, d0, d1, b1r, W2):
    return pl.pallas_call(
        _mid_body,
        grid=(N // _R,),
        in_specs=[pl.BlockSpec((_R, DH // 2), lambda i: (i, 0)),
                  pl.BlockSpec((_R, DH // 2), lambda i: (i, 0)),
                  pl.BlockSpec((_R, DEG_W), lambda i: (i, 0)),
                  pl.BlockSpec((_R, DEG_W), lambda i: (i, 0)),
                  pl.BlockSpec((1, DH), lambda i: (0, 0)),
                  pl.BlockSpec((DH, NCLS), lambda i: (0, 0))],
        out_specs=[pl.BlockSpec((_R, NCLS // 2), lambda i: (i, 0)),
                   pl.BlockSpec((_R, NCLS // 2), lambda i: (i, 0))],
        out_shape=[jax.ShapeDtypeStruct((N, NCLS // 2), jnp.float32)] * 2,
    )(a0, a1, d0, d1, b1r, W2)


def _out(a0, a1, d0, d1, b2r):
    return pl.pallas_call(
        _out_body,
        grid=(N // _R,),
        in_specs=[pl.BlockSpec((_R, NCLS // 2), lambda i: (i, 0)),
                  pl.BlockSpec((_R, NCLS // 2), lambda i: (i, 0)),
                  pl.BlockSpec((_R, DEG_W), lambda i: (i, 0)),
                  pl.BlockSpec((_R, DEG_W), lambda i: (i, 0)),
                  pl.BlockSpec((1, NCLS), lambda i: (0, 0))],
        out_specs=pl.BlockSpec((_R, NCLS), lambda i: (i, 0)),
        out_shape=jax.ShapeDtypeStruct((N, NCLS), jnp.float32),
    )(a0, a1, d0, d1, b2r)


_SC_KERNEL_CACHE = {}


def _sc_agg(width, with_deg):
    key = (width, with_deg)
    if key not in _SC_KERNEL_CACHE:
        _SC_KERNEL_CACHE[key] = _make_sc_agg(width, with_deg)
    return _SC_KERNEL_CACHE[key]


def kernel(x, edge_index, W1, b1, W2, b2):
    src2 = edge_index[0].reshape(E // BATCH, BATCH)
    dst2 = edge_index[1].reshape(E // BATCH, BATCH)
    z64 = jnp.zeros((N_PAD, DH // 2), jnp.float32)
    z32 = jnp.zeros((N_PAD, NCLS // 2), jnp.float32)
    zdeg = jnp.zeros((N_PAD, DEG_W), jnp.float32)
    ones = jnp.ones((BATCH, DEG_W), jnp.float32)

    h0, h1 = _mm1(x, W1)
    agg0, agg1, deg0, deg1 = _sc_agg(DH // 2, True)(
        h0, h1, src2, dst2, z64, zdeg, ones)
    g0, g1 = _mid(agg0, agg1, deg0, deg1, b1.reshape(1, DH), W2)
    s0, s1 = _sc_agg(NCLS // 2, False)(g0, g1, src2, dst2, z32)
    return _out(s0, s1, deg0, deg1, b2.reshape(1, NCLS))


# speedup vs baseline: 12.5066x; 1.1989x over previous
import functools

import jax
import jax.numpy as jnp
from jax import lax
from jax.experimental import pallas as pl
from jax.experimental.pallas import tpu as pltpu
from jax.experimental.pallas import tpu_sc as plsc

N = 10000
E = 320000
DF = 128
DH = 128
NCLS = 64

SC_CORES = 2
SC_TILES = 16
BATCH = 200
EDGES_PER_TILE = E // SC_TILES
BATCHES_PER_TILE = EDGES_PER_TILE // BATCH
N_PAD = 10240
ROWS_PER_TILE = N_PAD // SC_TILES
DEG_W = 8

_MESH = dict(core_axis_name="c", subcore_axis_name="s",
             num_cores=SC_CORES, num_subcores=SC_TILES)


NSLOT = 4


def _sc_agg_body(with_deg, width, *refs):
    if with_deg:
        (ei3, h0, h1, z_w, z_deg, ones_h,
         agg0, agg1, deg0, deg1, *rest) = refs
    else:
        (ei3, h0, h1, z_w,
         agg0, agg1, *rest) = refs
    src_i = rest[0:NSLOT]
    dst_i = rest[NSLOT:2 * NSLOT]
    rows = rest[2 * NSLOT:3 * NSLOT]
    sem_is = rest[3 * NSLOT:4 * NSLOT]
    sem_id = rest[4 * NSLOT:5 * NSLOT]
    sem_g = rest[5 * NSLOT:6 * NSLOT]
    sem_s = rest[6 * NSLOT:7 * NSLOT]
    accum = rest[7 * NSLOT]
    if with_deg:
        ones_v, dega = rest[7 * NSLOT + 1:]

    c = lax.axis_index("c")
    t = lax.axis_index("s")
    rsl = pl.ds(t * ROWS_PER_TILE, ROWS_PER_TILE)
    nb = BATCHES_PER_TILE
    half = nb // 2

    def start_src(j, s):
        off = (t * nb + j) * BATCH
        pltpu.async_copy(ei3.at[0, pl.ds(off, BATCH)], src_i[s], sem_is[s])

    def wait_src(s):
        pltpu.make_async_copy(ei3.at[0, pl.ds(0, BATCH)], src_i[s],
                              sem_is[s]).wait()

    def start_dst(j, s):
        off = (t * nb + j) * BATCH
        pltpu.async_copy(ei3.at[1, pl.ds(off, BATCH)], dst_i[s], sem_id[s])

    def wait_dst(s):
        pltpu.make_async_copy(ei3.at[1, pl.ds(0, BATCH)], dst_i[s],
                              sem_id[s]).wait()

    def start_gather(s):
        @pl.when(c == 0)
        def _():
            pltpu.async_copy(h0.at[src_i[s]], rows[s], sem_g[s])

        @pl.when(c == 1)
        def _():
            pltpu.async_copy(h1.at[src_i[s]], rows[s], sem_g[s])

    def wait_gather(s):
        pltpu.make_async_copy(h0.at[pl.ds(0, BATCH)], rows[s], sem_g[s]).wait()

    def deg_pred(j):
        return (j < half) == (c == 0)

    def start_scatter(j, s):
        pltpu.async_copy(rows[s], accum.at[dst_i[s]], sem_s[s], add=True)
        if with_deg:
            @pl.when(deg_pred(j))
            def _():
                pltpu.async_copy(ones_v, dega.at[dst_i[s]], sem_s[s], add=True)

    def wait_scatter(j, s):
        pltpu.make_async_copy(rows[s], accum.at[pl.ds(0, BATCH)],
                              sem_s[s]).wait()
        if with_deg:
            @pl.when(deg_pred(j))
            def _():
                pltpu.make_async_copy(ones_v, dega.at[pl.ds(0, BATCH)],
                                      sem_s[s]).wait()

    pltpu.sync_copy(z_w.at[rsl], accum.at[rsl])
    if with_deg:
        pltpu.sync_copy(z_deg.at[rsl], dega.at[rsl])
        pltpu.sync_copy(ones_h, ones_v)
    plsc.subcore_barrier()

    for s in range(NSLOT):
        start_src(s, s)
        start_dst(s, s)
    for s in range(NSLOT):
        wait_src(s)
        start_gather(s)

    def step(i, carry):
        for s in range(NSLOT):
            j = NSLOT * i + s
            wait_gather(s)

            @pl.when(i + 1 < nb // NSLOT)
            def _():
                start_src(j + NSLOT, s)

            wait_dst(s)
            start_scatter(j, s)

        @pl.when(i + 1 < nb // NSLOT)
        def _():
            for s in range(NSLOT):
                j = NSLOT * i + s
                wait_scatter(j, s)
                start_dst(j + NSLOT, s)
                wait_src(s)
                start_gather(s)

        return carry

    lax.fori_loop(0, nb // NSLOT, step, 0)
    for s in range(NSLOT):
        wait_scatter(nb - NSLOT + s, s)
    plsc.subcore_barrier()

    @pl.when(c == 0)
    def _():
        pltpu.sync_copy(accum.at[rsl], agg0.at[rsl])
        if with_deg:
            pltpu.sync_copy(dega.at[rsl], deg0.at[rsl])

    @pl.when(c == 1)
    def _():
        pltpu.sync_copy(accum.at[rsl], agg1.at[rsl])
        if with_deg:
            pltpu.sync_copy(dega.at[rsl], deg1.at[rsl])


def _make_sc_agg(width, with_deg):
    out_type = [jax.ShapeDtypeStruct((N_PAD, width), jnp.float32),
                jax.ShapeDtypeStruct((N_PAD, width), jnp.float32)]
    scratch = (
        [pltpu.VMEM((BATCH,), jnp.int32) for _ in range(2 * NSLOT)]
        + [pltpu.VMEM((BATCH, width), jnp.float32) for _ in range(NSLOT)]
        + [pltpu.SemaphoreType.DMA for _ in range(4 * NSLOT)]
        + [pltpu.VMEM_SHARED((N_PAD, width), jnp.float32)]
    )
    if with_deg:
        out_type += [jax.ShapeDtypeStruct((N_PAD, DEG_W), jnp.float32),
                     jax.ShapeDtypeStruct((N_PAD, DEG_W), jnp.float32)]
        scratch += [pltpu.VMEM((BATCH, DEG_W), jnp.float32),
                    pltpu.VMEM_SHARED((N_PAD, DEG_W), jnp.float32)]
    return pl.kernel(
        functools.partial(_sc_agg_body, with_deg, width),
        out_type=out_type,
        mesh=plsc.VectorSubcoreMesh(**_MESH),
        scratch_types=scratch,
        compiler_params=pltpu.CompilerParams(use_tc_tiling_on_sc=False),
    )


def _mm1_body(x_ref, w_ref, o0, o1):
    h = jnp.dot(x_ref[...], w_ref[...], preferred_element_type=jnp.float32)
    o0[...] = h[:, : DH // 2]
    o1[...] = h[:, DH // 2:]


def _mid_body(a0, a1, d0, d1, b1r, w2r, o0, o1):
    inv = 1.0 / jnp.maximum(d0[:, 0:1] + d1[:, 0:1], 1.0)
    h1a = jnp.maximum(a0[...] * inv + b1r[:, : DH // 2], 0.0)
    h1b = jnp.maximum(a1[...] * inv + b1r[:, DH // 2:], 0.0)
    h2 = (jnp.dot(h1a, w2r[0], preferred_element_type=jnp.float32)
          + jnp.dot(h1b, w2r[1], preferred_element_type=jnp.float32))
    o0[...] = h2[:, : NCLS // 2]
    o1[...] = h2[:, NCLS // 2:]


def _out_body(a0, a1, d0, d1, b2r, o):
    inv = 1.0 / jnp.maximum(d0[:, 0:1] + d1[:, 0:1], 1.0)
    za = a0[...] * inv + b2r[:, : NCLS // 2]
    zb = a1[...] * inv + b2r[:, NCLS // 2:]
    m = jnp.maximum(jnp.max(za, axis=1, keepdims=True),
                    jnp.max(zb, axis=1, keepdims=True))
    lse = jnp.log(jnp.sum(jnp.exp(za - m), axis=1, keepdims=True)
                  + jnp.sum(jnp.exp(zb - m), axis=1, keepdims=True))
    o[:, : NCLS // 2] = za - m - lse
    o[:, NCLS // 2:] = zb - m - lse


_R = 2000


def _mm1(x, W1):
    return pl.pallas_call(
        _mm1_body,
        grid=(N // _R,),
        in_specs=[pl.BlockSpec((_R, DF), lambda i: (i, 0)),
                  pl.BlockSpec((DF, DH), lambda i: (0, 0))],
        out_specs=[pl.BlockSpec((_R, DH // 2), lambda i: (i, 0)),
                   pl.BlockSpec((_R, DH // 2), lambda i: (i, 0))],
        out_shape=[jax.ShapeDtypeStruct((N, DH // 2), jnp.float32)] * 2,
    )(x, W1)


def _mid(a0, a1, d0, d1, b1r, W2):
    return pl.pallas_call(
        _mid_body,
        grid=(N // _R,),
        in_specs=[pl.BlockSpec((_R, DH // 2), lambda i: (i, 0)),
                  pl.BlockSpec((_R, DH // 2), lambda i: (i, 0)),
                  pl.BlockSpec((_R, DEG_W), lambda i: (i, 0)),
                  pl.BlockSpec((_R, DEG_W), lambda i: (i, 0)),
                  pl.BlockSpec((1, DH), lambda i: (0, 0)),
                  pl.BlockSpec((2, DH // 2, NCLS), lambda i: (0, 0, 0))],
        out_specs=[pl.BlockSpec((_R, NCLS // 2), lambda i: (i, 0)),
                   pl.BlockSpec((_R, NCLS // 2), lambda i: (i, 0))],
        out_shape=[jax.ShapeDtypeStruct((N, NCLS // 2), jnp.float32)] * 2,
    )(a0, a1, d0, d1, b1r, W2)


def _out(a0, a1, d0, d1, b2r):
    return pl.pallas_call(
        _out_body,
        grid=(N // _R,),
        in_specs=[pl.BlockSpec((_R, NCLS // 2), lambda i: (i, 0)),
                  pl.BlockSpec((_R, NCLS // 2), lambda i: (i, 0)),
                  pl.BlockSpec((_R, DEG_W), lambda i: (i, 0)),
                  pl.BlockSpec((_R, DEG_W), lambda i: (i, 0)),
                  pl.BlockSpec((1, NCLS), lambda i: (0, 0))],
        out_specs=pl.BlockSpec((_R, NCLS), lambda i: (i, 0)),
        out_shape=jax.ShapeDtypeStruct((N, NCLS), jnp.float32),
    )(a0, a1, d0, d1, b2r)


_SC_KERNEL_CACHE = {}


def _sc_agg(width, with_deg):
    key = (width, with_deg)
    if key not in _SC_KERNEL_CACHE:
        _SC_KERNEL_CACHE[key] = _make_sc_agg(width, with_deg)
    return _SC_KERNEL_CACHE[key]


def kernel(x, edge_index, W1, b1, W2, b2):
    ei3 = edge_index
    z64 = jnp.zeros((N_PAD, DH // 2), jnp.float32)
    z32 = jnp.zeros((N_PAD, NCLS // 2), jnp.float32)
    zdeg = jnp.zeros((N_PAD, DEG_W), jnp.float32)
    ones = jnp.ones((BATCH, DEG_W), jnp.float32)

    h0, h1 = _mm1(x, W1)
    agg0, agg1, deg0, deg1 = _sc_agg(DH // 2, True)(
        ei3, h0, h1, z64, zdeg, ones)
    g0, g1 = _mid(agg0, agg1, deg0, deg1, b1.reshape(1, DH),
                  W2.reshape(2, DH // 2, NCLS))
    s0, s1 = _sc_agg(NCLS // 2, False)(ei3, g0, g1, z32)
    return _out(s0, s1, deg0, deg1, b2.reshape(1, NCLS))
